# baseline (speedup 1.0000x reference)
import functools

import jax
import jax.numpy as jnp
from jax import lax
from jax.experimental import pallas as pl
from jax.experimental.pallas import tpu as pltpu
from jax.experimental.pallas import tpu_sc as plsc

N = 10000
E = 320000
F = 128
H = 128
C = 10
G = 64
HH = 64
NN = 10240
DUMP = N
ER = 2560
EP = ER * 128
RPW32 = ER // 32
RPS16 = ER // 16
ZR = NN // 16

_mesh = plsc.VectorSubcoreMesh(core_axis_name="c", subcore_axis_name="s")

NR16 = NN // 16


@functools.partial(
    pl.kernel,
    out_type=jax.ShapeDtypeStruct((2, NN), jnp.float32),
    mesh=_mesh,
    scratch_types=[
        pltpu.VMEM((RPW32, 128), jnp.int32),
        pltpu.VMEM((NN,), jnp.float32),
        pltpu.VMEM((NR16,), jnp.float32),
        pltpu.VMEM((NR16,), jnp.float32),
        pltpu.VMEM_SHARED((16, NN), jnp.float32),
    ],
    compiler_params=pltpu.CompilerParams(needs_layout_passes=False),
)
def _deg_kernel(dst_hbm, deg_hbm, dst_v, hist_v, tmp_v, acc_v, stage_sh):
    cid = lax.axis_index("c")
    sid = lax.axis_index("s")
    w = cid * 16 + sid
    z16 = jnp.zeros((16,), jnp.float32)

    @pl.loop(0, NN // 16)
    def _(i):
        hist_v[pl.ds(i * 16, 16)] = z16

    pltpu.sync_copy(dst_hbm.at[pl.ds(w * RPW32, RPW32)], dst_v)
    ones16 = jnp.ones((16,), jnp.float32)

    @pl.loop(0, RPW32)
    def _(i):
        for j in range(8):
            dvec = dst_v[i, pl.ds(j * 16, 16)]
            plsc.addupdate_scatter(hist_v, [dvec], ones16)

    pltpu.sync_copy(hist_v, stage_sh.at[sid])
    plsc.subcore_barrier()

    @pl.loop(0, NR16 // 16)
    def _(i):
        acc_v[pl.ds(i * 16, 16)] = z16

    for w2 in range(16):
        pltpu.sync_copy(stage_sh.at[w2, pl.ds(sid * NR16, NR16)], tmp_v)

        @pl.loop(0, NR16 // 16)
        def _(i):
            s = pl.ds(i * 16, 16)
            acc_v[s] = acc_v[s] + tmp_v[s]

    pltpu.sync_copy(acc_v, deg_hbm.at[cid, pl.ds(sid * NR16, NR16)])


RPW = ER // 32


CH = 40
A0 = 160


@functools.partial(
    pl.kernel,
    out_type=(
        jax.ShapeDtypeStruct((NN, H), jnp.float32),
        jax.ShapeDtypeStruct((NN, H), jnp.float32),
    ),
    mesh=_mesh,
    scratch_types=[
        pltpu.VMEM((CH, 128), jnp.int32),
        pltpu.VMEM((CH, 128), jnp.int32),
        pltpu.VMEM((128, H), jnp.float32),
        pltpu.VMEM((128, H), jnp.float32),
        pltpu.VMEM_SHARED((NN, H), jnp.float32),
        pltpu.SemaphoreType.DMA,
        pltpu.SemaphoreType.DMA,
    ],
    compiler_params=pltpu.CompilerParams(needs_layout_passes=False),
)
def _mp_kernel(tab, src2d, dst2d, out0, out1,
               src_v, dst_v, rows0_v, rows1_v, agg_sh, sem0, sem1):
    cid = lax.axis_index("c")
    sid = lax.axis_index("s")
    z16 = jnp.zeros((16,), jnp.float32)

    @pl.loop(0, 128)
    def _(i):
        for j in range(H // 16):
            rows0_v[i, pl.ds(j * 16, 16)] = z16

    for k in range(ZR // 128):
        pltpu.sync_copy(rows0_v, agg_sh.at[pl.ds(sid * ZR + k * 128, 128)])
    plsc.subcore_barrier()

    def work(base, nchunks):
      for c in range(nchunks):
        pltpu.sync_copy(src2d.at[pl.ds(base + c * CH, CH)], src_v)
        pltpu.sync_copy(dst2d.at[pl.ds(base + c * CH, CH)], dst_v)

        def gath(r, buf, sem):
            return pltpu.async_copy(tab.at[src_v.at[r]], buf, sem)

        gath(0, rows0_v, sem0)

        @pl.loop(0, CH // 2)
        def _(j):
            r0 = 2 * j
            r1 = r0 + 1
            pltpu.make_async_copy(tab.at[src_v.at[r0]], rows0_v,
                                  sem0).wait()
            gath(r1, rows1_v, sem1)
            pltpu.sync_copy(rows0_v, agg_sh.at[dst_v.at[r0]], add=True)
            pltpu.make_async_copy(tab.at[src_v.at[r1]], rows1_v,
                                  sem1).wait()

            @pl.when(j < CH // 2 - 1)
            def _():
                gath(r0 + 2, rows0_v, sem0)

            pltpu.sync_copy(rows1_v, agg_sh.at[dst_v.at[r1]], add=True)

    @pl.when(cid == 0)
    def _():
        work(sid * (2 * RPW), A0 // CH)

    @pl.when(cid == 1)
    def _():
        work(sid * (2 * RPW) + A0, (2 * RPW - A0) // CH)

    plsc.subcore_barrier()

    def copy_out(out):
        for k in range(ZR // 128):
            pltpu.sync_copy(agg_sh.at[pl.ds(sid * ZR + k * 128, 128)],
                            rows0_v)
            pltpu.sync_copy(rows0_v, out.at[pl.ds(sid * ZR + k * 128, 128)])

    @pl.when(cid == 0)
    def _():
        copy_out(out0)

    @pl.when(cid == 1)
    def _():
        copy_out(out1)


BN = 2048


def _lin1_body(x_ref, w_ref, deg_ref, y_ref, dinv_ref):
    xw = jnp.dot(x_ref[...], w_ref[...], preferred_element_type=jnp.float32)
    deg = jnp.sum(deg_ref[...], axis=0) + 1.0
    dinv = lax.rsqrt(deg)
    y_ref[...] = xw * dinv
    dinv_ref[...] = dinv


_lin1 = pl.pallas_call(
    _lin1_body,
    grid=(NN // BN,),
    in_specs=[
        pl.BlockSpec((BN, F), lambda i: (i, 0)),
        pl.BlockSpec((F, H), lambda i: (0, 0)),
        pl.BlockSpec((2, BN, 1), lambda i: (0, i, 0)),
    ],
    out_specs=[
        pl.BlockSpec((BN, H), lambda i: (i, 0)),
        pl.BlockSpec((BN, 1), lambda i: (i, 0)),
    ],
    out_shape=[
        jax.ShapeDtypeStruct((NN, H), jnp.float32),
        jax.ShapeDtypeStruct((NN, 1), jnp.float32),
    ],
)


def _lin2_body(a0, a1, y1, dinv, b1_ref, w2_ref, o_ref):
    agg = a0[...] + a1[...]
    dv = dinv[...]
    h = jnp.maximum(dv * (agg + y1[...]) + b1_ref[...], 0.0)
    xw = jnp.dot(h, w2_ref[...], preferred_element_type=jnp.float32)
    o_ref[...] = xw * dv


_lin2 = pl.pallas_call(
    _lin2_body,
    grid=(NN // BN,),
    in_specs=[
        pl.BlockSpec((BN, H), lambda i: (i, 0)),
        pl.BlockSpec((BN, H), lambda i: (i, 0)),
        pl.BlockSpec((BN, H), lambda i: (i, 0)),
        pl.BlockSpec((BN, 1), lambda i: (i, 0)),
        pl.BlockSpec((1, H), lambda i: (0, 0)),
        pl.BlockSpec((H, H), lambda i: (0, 0)),
    ],
    out_specs=pl.BlockSpec((BN, H), lambda i: (i, 0)),
    out_shape=jax.ShapeDtypeStruct((NN, H), jnp.float32),
)


def _pool_body(a0, a1, y2, dinv, b2_ref, batch_ref, batcht_ref,
               linw_ref, linb_ref, out_ref, mx_ref, sm_ref, cnt_ref):
    i = pl.program_id(0)
    agg = a0[...] + a1[...]
    h = jnp.maximum(dinv[...] * (agg + y2[...]) + b2_ref[...], 0.0)

    @pl.when(i == 0)
    def _():
        mx_ref[...] = jnp.full((G + 8, H), -jnp.inf, jnp.float32)
        sm_ref[...] = jnp.zeros((G, H), jnp.float32)
        cnt_ref[...] = jnp.zeros((G, 1), jnp.float32)

    bt = batcht_ref[...]
    onehot_t = (bt == lax.broadcasted_iota(jnp.int32, (G, 1), 0)).astype(
        jnp.float32)
    sm_ref[...] += jnp.dot(onehot_t, h, preferred_element_type=jnp.float32)
    cnt_ref[...] += jnp.sum(onehot_t, axis=1, keepdims=True)

    b = batch_ref[...]
    g0 = b[0, 0]
    g1 = b[BN - 1, 0]

    def mbody(g, carry):
        mask = b == g
        vals = jnp.where(mask, h, -jnp.inf)
        m = jnp.max(vals, axis=0)[None, :]
        mx_ref[pl.ds(g, 1), :] = jnp.maximum(mx_ref[pl.ds(g, 1), :], m)
        return carry

    lax.fori_loop(g0, g1 + 1, mbody, 0)

    @pl.when(i == pl.num_programs(0) - 1)
    def _():
        cnt = jnp.maximum(cnt_ref[...], 1.0)
        mean = sm_ref[...] / cnt
        mx = mx_ref[:G, :]
        z = (jnp.dot(mx, linw_ref[:H, :], preferred_element_type=jnp.float32)
             + jnp.dot(mean, linw_ref[H:, :],
                       preferred_element_type=jnp.float32)
             + linb_ref[...])
        zm = jnp.max(z, axis=1, keepdims=True)
        lse = zm + jnp.log(jnp.sum(jnp.exp(z - zm), axis=1, keepdims=True))
        out_ref[...] = z - lse


_pool = pl.pallas_call(
    _pool_body,
    grid=(NN // BN,),
    in_specs=[
        pl.BlockSpec((BN, H), lambda i: (i, 0)),
        pl.BlockSpec((BN, H), lambda i: (i, 0)),
        pl.BlockSpec((BN, H), lambda i: (i, 0)),
        pl.BlockSpec((BN, 1), lambda i: (i, 0)),
        pl.BlockSpec((1, H), lambda i: (0, 0)),
        pl.BlockSpec((BN, 1), lambda i: (i, 0)),
        pl.BlockSpec((1, BN), lambda i: (0, i)),
        pl.BlockSpec((2 * H, C), lambda i: (0, 0)),
        pl.BlockSpec((1, C), lambda i: (0, 0)),
    ],
    out_specs=pl.BlockSpec((G, C), lambda i: (0, 0)),
    out_shape=jax.ShapeDtypeStruct((G, C), jnp.float32),
    scratch_shapes=[
        pltpu.VMEM((G + 8, H), jnp.float32),
        pltpu.VMEM((G, H), jnp.float32),
        pltpu.VMEM((G, 1), jnp.float32),
    ],
)


def kernel(x, edge_index, batch, W1, b1, W2, b2, lin_W, lin_b):
    src = edge_index[0]
    dst = edge_index[1]
    pad = EP - E
    srcp = jnp.concatenate(
        [src, jnp.zeros((pad,), jnp.int32)]).reshape(ER, 128)
    dstp = jnp.concatenate(
        [dst, jnp.full((pad,), DUMP, jnp.int32)]).reshape(ER, 128)
    x_p = jnp.pad(x, ((0, NN - N), (0, 0)))
    batch_p = jnp.concatenate([batch, jnp.full((NN - N,), G, jnp.int32)])
    deg2 = _deg_kernel(dstp).reshape(2, NN, 1)
    y1, dinv = _lin1(x_p, W1, deg2)
    a10, a11 = _mp_kernel(y1, srcp, dstp)
    y2 = _lin2(a10, a11, y1, dinv, b1.reshape(1, H), W2)
    a20, a21 = _mp_kernel(y2, srcp, dstp)
    out = _pool(a20, a21, y2, dinv, b2.reshape(1, H),
                batch_p[:, None], batch_p[None, :], lin_W,
                lin_b.reshape(1, C))
    return out


# speedup vs baseline: 1.6816x; 1.6816x over previous
import functools

import jax
import jax.numpy as jnp
from jax import lax
from jax.experimental import pallas as pl
from jax.experimental.pallas import tpu as pltpu
from jax.experimental.pallas import tpu_sc as plsc

N = 10000
E = 320000
F = 128
H = 128
C = 10
G = 64
HH = 64
NN = 10240
DUMP = N
ER = 2560
EP = ER * 128
RPW32 = ER // 32
ZR = NN // 16
EW = 64
ER64 = EP // EW

_mesh = plsc.VectorSubcoreMesh(core_axis_name="c", subcore_axis_name="s")

NR16 = NN // 16


@functools.partial(
    pl.kernel,
    out_type=jax.ShapeDtypeStruct((2, NN), jnp.float32),
    mesh=_mesh,
    scratch_types=[
        pltpu.VMEM((2 * RPW32, EW), jnp.int32),
        pltpu.VMEM((NN,), jnp.float32),
        pltpu.VMEM((NR16,), jnp.float32),
        pltpu.VMEM((NR16,), jnp.float32),
        pltpu.VMEM_SHARED((16, NN), jnp.float32),
    ],
    compiler_params=pltpu.CompilerParams(needs_layout_passes=False),
)
def _deg_kernel(dst_hbm, deg_hbm, dst_v, hist_v, tmp_v, acc_v, stage_sh):
    cid = lax.axis_index("c")
    sid = lax.axis_index("s")
    w = cid * 16 + sid
    z16 = jnp.zeros((16,), jnp.float32)

    @pl.loop(0, NN // 16)
    def _(i):
        hist_v[pl.ds(i * 16, 16)] = z16

    pltpu.sync_copy(dst_hbm.at[pl.ds(w * (2 * RPW32), 2 * RPW32)], dst_v)
    ones16 = jnp.ones((16,), jnp.float32)

    @pl.loop(0, 2 * RPW32)
    def _(i):
        for j in range(EW // 16):
            dvec = dst_v[i, pl.ds(j * 16, 16)]
            plsc.addupdate_scatter(hist_v, [dvec], ones16)

    pltpu.sync_copy(hist_v, stage_sh.at[sid])
    plsc.subcore_barrier()

    @pl.loop(0, NR16 // 16)
    def _(i):
        acc_v[pl.ds(i * 16, 16)] = z16

    for w2 in range(16):
        pltpu.sync_copy(stage_sh.at[w2, pl.ds(sid * NR16, NR16)], tmp_v)

        @pl.loop(0, NR16 // 16)
        def _(i):
            s = pl.ds(i * 16, 16)
            acc_v[s] = acc_v[s] + tmp_v[s]

    pltpu.sync_copy(acc_v, deg_hbm.at[cid, pl.ds(sid * NR16, NR16)])


RPW = ER // 32


SPAN = ER64 // 16
A0R = 240
CH = 40
NB = 4


@functools.partial(
    pl.kernel,
    out_type=(
        jax.ShapeDtypeStruct((NN, H), jnp.float32),
        jax.ShapeDtypeStruct((NN, H), jnp.float32),
    ),
    mesh=_mesh,
    scratch_types=[
        pltpu.VMEM((CH, EW), jnp.int32),
        pltpu.VMEM((CH, EW), jnp.int32),
        pltpu.VMEM((NB, EW, H), jnp.float32),
        pltpu.VMEM_SHARED((NN, H), jnp.float32),
        [pltpu.SemaphoreType.DMA] * NB,
        [pltpu.SemaphoreType.DMA] * NB,
    ],
    compiler_params=pltpu.CompilerParams(needs_layout_passes=False),
)
def _mp_kernel(tab, src2d, dst2d, out0, out1,
               src_v, dst_v, rows_v, agg_sh, gsems, ssems):
    cid = lax.axis_index("c")
    sid = lax.axis_index("s")
    z16 = jnp.zeros((16,), jnp.float32)

    @pl.loop(0, EW)
    def _(i):
        for j in range(H // 16):
            rows_v[0, i, pl.ds(j * 16, 16)] = z16

    for k in range(ZR // EW):
        pltpu.sync_copy(rows_v.at[0],
                        agg_sh.at[pl.ds(sid * ZR + k * EW, EW)])
    plsc.subcore_barrier()

    def gath_start(r, b):
        pltpu.async_copy(tab.at[src_v.at[r]], rows_v.at[b], gsems[b])

    def gath_wait(r, b):
        pltpu.make_async_copy(tab.at[src_v.at[r]], rows_v.at[b],
                              gsems[b]).wait()

    def scat_start(r, b):
        pltpu.async_copy(rows_v.at[b], agg_sh.at[dst_v.at[r]], ssems[b],
                         add=True)

    def scat_wait(b):
        pltpu.make_async_copy(rows_v.at[b], agg_sh.at[dst_v.at[0]],
                              ssems[b]).wait()

    def work(base, nchunks):
      for c in range(nchunks):
        pltpu.sync_copy(src2d.at[pl.ds(base + c * CH, CH)], src_v)
        pltpu.sync_copy(dst2d.at[pl.ds(base + c * CH, CH)], dst_v)
        for b in range(NB - 1):
            gath_start(b, b)

        @pl.loop(0, CH // NB)
        def _(i):
            for k in range(NB):
                j = NB * i + k
                gath_wait(j, k)
                scat_start(j, k)
                bn = (k + NB - 1) % NB
                if k == 0:
                    @pl.when(i >= 1)
                    def _():
                        scat_wait(bn)

                    gath_start(j + NB - 1, bn)
                else:
                    @pl.when(i < CH // NB - 1)
                    def _():
                        scat_wait(bn)
                        gath_start(j + NB - 1, bn)

        for b in range(NB):
            scat_wait(b)

    @pl.when(cid == 0)
    def _():
        work(sid * SPAN, A0R // CH)

    @pl.when(cid == 1)
    def _():
        work(sid * SPAN + A0R, (SPAN - A0R) // CH)

    plsc.subcore_barrier()

    def copy_out(out):
        for k in range(ZR // EW):
            pltpu.sync_copy(agg_sh.at[pl.ds(sid * ZR + k * EW, EW)],
                            rows_v.at[k % NB])
            pltpu.sync_copy(rows_v.at[k % NB],
                            out.at[pl.ds(sid * ZR + k * EW, EW)])

    @pl.when(cid == 0)
    def _():
        copy_out(out0)

    @pl.when(cid == 1)
    def _():
        copy_out(out1)


BN = 2048


def _lin1_body(x_ref, w_ref, deg_ref, y_ref, dinv_ref):
    xw = jnp.dot(x_ref[...], w_ref[...], preferred_element_type=jnp.float32)
    deg = jnp.sum(deg_ref[...], axis=0) + 1.0
    dinv = lax.rsqrt(deg)
    y_ref[...] = xw * dinv
    dinv_ref[...] = dinv


_lin1 = pl.pallas_call(
    _lin1_body,
    grid=(NN // BN,),
    in_specs=[
        pl.BlockSpec((BN, F), lambda i: (i, 0)),
        pl.BlockSpec((F, H), lambda i: (0, 0)),
        pl.BlockSpec((2, BN, 1), lambda i: (0, i, 0)),
    ],
    out_specs=[
        pl.BlockSpec((BN, H), lambda i: (i, 0)),
        pl.BlockSpec((BN, 1), lambda i: (i, 0)),
    ],
    out_shape=[
        jax.ShapeDtypeStruct((NN, H), jnp.float32),
        jax.ShapeDtypeStruct((NN, 1), jnp.float32),
    ],
)


def _lin2_body(a0, a1, y1, dinv, b1_ref, w2_ref, o_ref):
    agg = a0[...] + a1[...]
    dv = dinv[...]
    h = jnp.maximum(dv * (agg + y1[...]) + b1_ref[...], 0.0)
    xw = jnp.dot(h, w2_ref[...], preferred_element_type=jnp.float32)
    o_ref[...] = xw * dv


_lin2 = pl.pallas_call(
    _lin2_body,
    grid=(NN // BN,),
    in_specs=[
        pl.BlockSpec((BN, H), lambda i: (i, 0)),
        pl.BlockSpec((BN, H), lambda i: (i, 0)),
        pl.BlockSpec((BN, H), lambda i: (i, 0)),
        pl.BlockSpec((BN, 1), lambda i: (i, 0)),
        pl.BlockSpec((1, H), lambda i: (0, 0)),
        pl.BlockSpec((H, H), lambda i: (0, 0)),
    ],
    out_specs=pl.BlockSpec((BN, H), lambda i: (i, 0)),
    out_shape=jax.ShapeDtypeStruct((NN, H), jnp.float32),
)


def _pool_body(a0, a1, y2, dinv, b2_ref, batch_ref, batcht_ref,
               linw_ref, linb_ref, out_ref, mx_ref, sm_ref, cnt_ref):
    i = pl.program_id(0)
    agg = a0[...] + a1[...]
    h = jnp.maximum(dinv[...] * (agg + y2[...]) + b2_ref[...], 0.0)

    @pl.when(i == 0)
    def _():
        mx_ref[...] = jnp.full((G + 8, H), -jnp.inf, jnp.float32)
        sm_ref[...] = jnp.zeros((G, H), jnp.float32)
        cnt_ref[...] = jnp.zeros((G, 1), jnp.float32)

    bt = batcht_ref[...]
    onehot_t = (bt == lax.broadcasted_iota(jnp.int32, (G, 1), 0)).astype(
        jnp.float32)
    sm_ref[...] += jnp.dot(onehot_t, h, preferred_element_type=jnp.float32)
    cnt_ref[...] += jnp.sum(onehot_t, axis=1, keepdims=True)

    b = batch_ref[...]
    g0 = b[0, 0]
    g1 = b[BN - 1, 0]

    def mbody(g, carry):
        mask = b == g
        vals = jnp.where(mask, h, -jnp.inf)
        m = jnp.max(vals, axis=0)[None, :]
        mx_ref[pl.ds(g, 1), :] = jnp.maximum(mx_ref[pl.ds(g, 1), :], m)
        return carry

    lax.fori_loop(g0, g1 + 1, mbody, 0)

    @pl.when(i == pl.num_programs(0) - 1)
    def _():
        cnt = jnp.maximum(cnt_ref[...], 1.0)
        mean = sm_ref[...] / cnt
        mx = mx_ref[:G, :]
        z = (jnp.dot(mx, linw_ref[:H, :], preferred_element_type=jnp.float32)
             + jnp.dot(mean, linw_ref[H:, :],
                       preferred_element_type=jnp.float32)
             + linb_ref[...])
        zm = jnp.max(z, axis=1, keepdims=True)
        lse = zm + jnp.log(jnp.sum(jnp.exp(z - zm), axis=1, keepdims=True))
        out_ref[...] = z - lse


_pool = pl.pallas_call(
    _pool_body,
    grid=(NN // BN,),
    in_specs=[
        pl.BlockSpec((BN, H), lambda i: (i, 0)),
        pl.BlockSpec((BN, H), lambda i: (i, 0)),
        pl.BlockSpec((BN, H), lambda i: (i, 0)),
        pl.BlockSpec((BN, 1), lambda i: (i, 0)),
        pl.BlockSpec((1, H), lambda i: (0, 0)),
        pl.BlockSpec((BN, 1), lambda i: (i, 0)),
        pl.BlockSpec((1, BN), lambda i: (0, i)),
        pl.BlockSpec((2 * H, C), lambda i: (0, 0)),
        pl.BlockSpec((1, C), lambda i: (0, 0)),
    ],
    out_specs=pl.BlockSpec((G, C), lambda i: (0, 0)),
    out_shape=jax.ShapeDtypeStruct((G, C), jnp.float32),
    scratch_shapes=[
        pltpu.VMEM((G + 8, H), jnp.float32),
        pltpu.VMEM((G, H), jnp.float32),
        pltpu.VMEM((G, 1), jnp.float32),
    ],
)


def kernel(x, edge_index, batch, W1, b1, W2, b2, lin_W, lin_b):
    src = edge_index[0]
    dst = edge_index[1]
    pad = EP - E
    srcp = jnp.concatenate(
        [src, jnp.zeros((pad,), jnp.int32)]).reshape(ER64, EW)
    dstp = jnp.concatenate(
        [dst, jnp.full((pad,), DUMP, jnp.int32)]).reshape(ER64, EW)
    x_p = jnp.pad(x, ((0, NN - N), (0, 0)))
    batch_p = jnp.concatenate([batch, jnp.full((NN - N,), G, jnp.int32)])
    deg2 = _deg_kernel(dstp).reshape(2, NN, 1)
    y1, dinv = _lin1(x_p, W1, deg2)
    a10, a11 = _mp_kernel(y1, srcp, dstp)
    y2 = _lin2(a10, a11, y1, dinv, b1.reshape(1, H), W2)
    a20, a21 = _mp_kernel(y2, srcp, dstp)
    out = _pool(a20, a21, y2, dinv, b2.reshape(1, H),
                batch_p[:, None], batch_p[None, :], lin_W,
                lin_b.reshape(1, C))
    return out
